# baseline (speedup 1.0000x reference)
import functools

import jax
import jax.numpy as jnp
from jax import lax
from jax.experimental import pallas as pl
from jax.experimental.pallas import tpu as pltpu
from jax.experimental.pallas import tpu_sc as plsc

_B = 16384
_S = 200
_D = 64
_ROW = _S * _D
_NC = 2
_NS = 16
_NW = _NC * _NS
_PER_W = _B // _NW
_CHUNK = 8
_NDMA = _PER_W // _CHUNK


def _make_sc_broadcast():
    mesh = plsc.VectorSubcoreMesh(core_axis_name="c", subcore_axis_name="s")

    @functools.partial(
        pl.kernel,
        mesh=mesh,
        out_type=jax.ShapeDtypeStruct((_B * _ROW,), jnp.float32),
        scratch_types=[
            pltpu.VMEM((_CHUNK * _ROW,), jnp.float32),
            pltpu.SemaphoreType.DMA,
        ],
    )
    def body(pos_embed_hbm, out_hbm, buf, sem):
        wid = lax.axis_index("s") * _NC + lax.axis_index("c")
        base = wid * _PER_W * _ROW
        for i in range(_CHUNK):
            pltpu.sync_copy(
                pos_embed_hbm.at[pl.ds(0, _ROW)],
                buf.at[pl.ds(i * _ROW, _ROW)],
            )
        copies = [
            pltpu.async_copy(
                buf,
                out_hbm.at[pl.ds(base + j * _CHUNK * _ROW, _CHUNK * _ROW)],
                sem,
            )
            for j in range(_NDMA)
        ]
        for c in copies:
            c.wait()

    return body


_sc_broadcast = _make_sc_broadcast()


def kernel(x, pos_embed):
    del x
    flat = _sc_broadcast(pos_embed.reshape(-1))
    return flat.reshape(_B, _S, _D)


# speedup vs baseline: 2.1396x; 2.1396x over previous
import functools

import jax
import jax.numpy as jnp
from jax import lax
from jax.experimental import pallas as pl
from jax.experimental.pallas import tpu as pltpu
from jax.experimental.pallas import tpu_sc as plsc

_B = 16384
_S = 200
_D = 64
_ROW = _S * _D // 128
_NC = 2
_NS = 16
_NW = _NC * _NS
_PER_W = _B // _NW
_CHUNK = 8
_NDMA = _PER_W // _CHUNK


def _make_sc_broadcast():
    mesh = plsc.VectorSubcoreMesh(core_axis_name="c", subcore_axis_name="s")

    @functools.partial(
        pl.kernel,
        mesh=mesh,
        out_type=jax.ShapeDtypeStruct((_B, _ROW, 128), jnp.float32),
        scratch_types=[
            pltpu.VMEM((_CHUNK, _ROW, 128), jnp.float32),
            pltpu.SemaphoreType.DMA,
        ],
    )
    def body(emb_hbm, out_hbm, buf, sem):
        wid = lax.axis_index("s") * _NC + lax.axis_index("c")
        base = wid * _PER_W
        for i in range(_CHUNK):
            pltpu.sync_copy(emb_hbm, buf.at[i])
        copies = [
            pltpu.async_copy(
                buf, out_hbm.at[pl.ds(base + j * _CHUNK, _CHUNK)], sem
            )
            for j in range(_NDMA)
        ]
        for c in copies:
            c.wait()

    return body


_sc_broadcast = _make_sc_broadcast()


def kernel(x, pos_embed):
    del x
    emb = jax.lax.slice(pos_embed, (0, 0), (_S, _D)).reshape(_ROW, 128)
    return _sc_broadcast(emb).reshape(_B, _S, _D)


# speedup vs baseline: 7.5641x; 3.5353x over previous
import functools

import jax
import jax.numpy as jnp
from jax import lax
from jax.experimental import pallas as pl
from jax.experimental.pallas import tpu as pltpu
from jax.experimental.pallas import tpu_sc as plsc

_B = 16384
_S = 200
_D = 64
_ROWS = _S * _D
_EROWS = _ROWS // 128
_NC = 2
_NS = 16
_RPT = _ROWS // _NS
_CPC = _B // _NC
_CB = 128
_NDMA = _CPC // _CB


def _seed_body(e_ref, o_ref):
    o_ref[...] = jnp.broadcast_to(
        e_ref[...][:, :, None], (_EROWS, 128, _CB)
    )


_seed_call = pl.pallas_call(
    _seed_body,
    out_shape=jax.ShapeDtypeStruct((_EROWS, 128, _CB), jnp.float32),
)


def _make_sc_fanout():
    mesh = plsc.VectorSubcoreMesh(core_axis_name="c", subcore_axis_name="s")

    @functools.partial(
        pl.kernel,
        mesh=mesh,
        out_type=jax.ShapeDtypeStruct((_ROWS, _B), jnp.float32),
        scratch_types=[
            pltpu.VMEM((_RPT, _CB), jnp.float32),
            pltpu.SemaphoreType.DMA,
        ],
    )
    def body(seed_hbm, out_hbm, buf, sem):
        cid = lax.axis_index("c")
        tid = lax.axis_index("s")
        row0 = tid * _RPT
        col0 = cid * _CPC
        pltpu.sync_copy(seed_hbm.at[pl.ds(row0, _RPT)], buf)
        copies = [
            pltpu.async_copy(
                buf,
                out_hbm.at[pl.ds(row0, _RPT), pl.ds(col0 + j * _CB, _CB)],
                sem,
            )
            for j in range(_NDMA)
        ]
        for c in copies:
            c.wait()

    return body


_sc_fanout = _make_sc_fanout()


def kernel(x, pos_embed):
    del x
    emb = pos_embed.reshape(-1)[: _ROWS].reshape(_EROWS, 128)
    seed = _seed_call(emb).reshape(_ROWS, _CB)
    flat = _sc_fanout(seed)
    return flat.reshape(_S, _D, _B).transpose(2, 0, 1)
